# Initial kernel scaffold; baseline (speedup 1.0000x reference)
import jax
import jax.numpy as jnp
from jax.experimental import pallas as pl


def kernel(output):
    raise NotImplementedError("write your pallas kernel here")


# speedup vs baseline: 49.6673x; 49.6673x over previous
import jax
import jax.numpy as jnp
from jax import lax
from jax.experimental import pallas as pl
from jax.experimental.pallas import tpu as pltpu

_B = 128
_THRESH = 0.5


def _nms_body(st_ref, en_ref, pk_ref, ht_ref,
              stT_ref, enT_ref, pkT_ref, htT_ref,
              out_ref, keep_ref):
    k = pl.program_id(0)
    B = _B

    cst = stT_ref[0]
    cen = enT_ref[0]
    cpk = pkT_ref[0]
    cht = htT_ref[0]
    carea = (cen - cst) * cht

    def scores(sst, sen, spk, sht):
        inter_start = jnp.maximum(cst, sst)
        inter_end = jnp.minimum(cen, sen)
        inter_len = jnp.maximum(inter_end - inter_start, 0.0)
        inter_h = jnp.minimum(cht, sht)
        inter_area = inter_len * inter_h
        sarea = (sen - sst) * sht
        union_area = sarea + carea - inter_area
        iou = inter_area / union_area
        pd = jnp.abs(spk - cpk)
        union_start = jnp.minimum(cst, sst)
        union_end = jnp.maximum(cen, sen)
        ud = jnp.abs(union_end - union_start)
        return iou - pd / ud

    def cross(m, supp):
        sst = st_ref[pl.ds(m, 1), :]
        sen = en_ref[pl.ds(m, 1), :]
        spk = pk_ref[pl.ds(m, 1), :]
        sht = ht_ref[pl.ds(m, 1), :]
        sc = scores(sst, sen, spk, sht)
        krow = keep_ref[pl.ds(m, 1), :]
        hit = (sc > _THRESH) & (krow > 0.5)
        hitv = jnp.any(hit, axis=1, keepdims=True).astype(jnp.float32)
        return jnp.maximum(supp, hitv)

    supp = lax.fori_loop(0, k, cross, jnp.zeros((B, 1), jnp.float32))

    dst = st_ref[pl.ds(k, 1), :]
    den = en_ref[pl.ds(k, 1), :]
    dpk = pk_ref[pl.ds(k, 1), :]
    dht = ht_ref[pl.ds(k, 1), :]
    sc = scores(dst, den, dpk, dht)
    jlane = lax.broadcasted_iota(jnp.int32, (B, B), 1)
    bsub = lax.broadcasted_iota(jnp.int32, (B, B), 0)
    mat = (sc > _THRESH) & (jlane < bsub)
    eye = (jlane == bsub).astype(jnp.float32)

    def t(col):
        return lax.dot_general(col, eye, (((0,), (0,)), ((), ())),
                               preferred_element_type=jnp.float32)

    ext = 1.0 - supp

    def fp_cond(c):
        return c[1] > 0

    def fp_body(c):
        kc, _ = c
        krow = t(kc)
        sup2 = jnp.any(mat & (krow > 0.5), axis=1, keepdims=True)
        new = jnp.where(sup2, 0.0, ext)
        changed = jnp.sum(jnp.abs(new - kc)).astype(jnp.int32)
        return new, changed

    kc, _ = lax.while_loop(fp_cond, fp_body, (ext, jnp.int32(1)))

    out_ref[0] = kc
    keep_ref[pl.ds(k, 1), :] = t(kc)


def _nms_keep(st, en, pk, ht, interpret=False):
    K, B = st.shape
    stT = st.reshape(K, B, 1)
    enT = en.reshape(K, B, 1)
    pkT = pk.reshape(K, B, 1)
    htT = ht.reshape(K, B, 1)
    full = pl.BlockSpec((K, B), lambda k: (0, 0))
    colb = pl.BlockSpec((1, B, 1), lambda k: (k, 0, 0))
    keep = pl.pallas_call(
        _nms_body,
        grid=(K,),
        in_specs=[full, full, full, full, colb, colb, colb, colb],
        out_specs=colb,
        out_shape=jax.ShapeDtypeStruct((K, B, 1), jnp.float32),
        scratch_shapes=[pltpu.VMEM((K, B), jnp.float32)],
        interpret=interpret,
    )(st, en, pk, ht, stT, enT, pkT, htT)
    return keep.reshape(K * B)


def _run(output, interpret=False):
    n, c = output.shape
    order = jnp.argsort(-output[:, 0])
    boxes = output[order]
    K = -(-n // _B)
    npad = K * _B
    pad = jnp.zeros((npad - n, c), jnp.float32)
    pad = pad.at[:, 1].set(1e9).at[:, 2].set(2e9).at[:, 3].set(1.5e9)
    pad = pad.at[:, 4].set(1.0)
    bp = jnp.concatenate([boxes, pad], axis=0)
    st = bp[:, 1].reshape(K, _B)
    en = bp[:, 2].reshape(K, _B)
    pk = bp[:, 3].reshape(K, _B)
    ht = bp[:, 4].reshape(K, _B)
    keep = _nms_keep(st, en, pk, ht, interpret=interpret)[:n]
    return boxes[:, 1:] * keep[:, None]


def kernel(output):
    return _run(output)


# speedup vs baseline: 100.4660x; 2.0228x over previous
import jax
import jax.numpy as jnp
from jax import lax
from jax.experimental import pallas as pl
from jax.experimental.pallas import tpu as pltpu

_B = 128
_THRESH = 0.5


def _nms_body(st_ref, en_ref, pk_ref, ht_ref,
              stT_ref, enT_ref, pkT_ref, htT_ref,
              out_ref,
              cst_ref, cen_ref, cpk_ref, cht_ref, cur_ref):
    k = pl.program_id(0)
    B = _B

    @pl.when(k == 0)
    def _init():
        zero = jnp.zeros_like(cst_ref)
        cst_ref[...] = zero
        cen_ref[...] = zero
        cpk_ref[...] = zero
        cht_ref[...] = zero
        cur_ref[0] = 0

    cst = stT_ref[0]
    cen = enT_ref[0]
    cpk = pkT_ref[0]
    cht = htT_ref[0]
    carea = (cen - cst) * cht

    def scores(sst, sen, spk, sht):
        inter_start = jnp.maximum(cst, sst)
        inter_end = jnp.minimum(cen, sen)
        inter_len = jnp.maximum(inter_end - inter_start, 0.0)
        inter_h = jnp.minimum(cht, sht)
        inter_area = inter_len * inter_h
        sarea = (sen - sst) * sht
        union_area = sarea + carea - inter_area
        iou = inter_area / union_area
        pd = jnp.abs(spk - cpk)
        union_start = jnp.minimum(cst, sst)
        union_end = jnp.maximum(cen, sen)
        ud = jnp.abs(union_end - union_start)
        return iou - pd / ud

    cur = cur_ref[0]

    def cross(r, supp):
        sst = cst_ref[pl.ds(r, 1), :]
        sen = cen_ref[pl.ds(r, 1), :]
        spk = cpk_ref[pl.ds(r, 1), :]
        sht = cht_ref[pl.ds(r, 1), :]
        sc = scores(sst, sen, spk, sht)
        hitv = jnp.any(sc > _THRESH, axis=1, keepdims=True).astype(jnp.float32)
        return jnp.maximum(supp, hitv)

    nrows = (cur + B - 1) // B
    supp = lax.fori_loop(0, nrows, cross, jnp.zeros((B, 1), jnp.float32))

    dst = st_ref[pl.ds(k, 1), :]
    den = en_ref[pl.ds(k, 1), :]
    dpk = pk_ref[pl.ds(k, 1), :]
    dht = ht_ref[pl.ds(k, 1), :]
    sc = scores(dst, den, dpk, dht)
    jlane = lax.broadcasted_iota(jnp.int32, (B, B), 1)
    bsub = lax.broadcasted_iota(jnp.int32, (B, B), 0)
    mat = (sc > _THRESH) & (jlane < bsub)
    eye = (jlane == bsub).astype(jnp.float32)
    ltri = (jlane <= bsub).astype(jnp.float32)

    def t(col):
        return lax.dot_general(col, eye, (((0,), (0,)), ((), ())),
                               preferred_element_type=jnp.float32, precision=lax.Precision.HIGHEST)

    ext = 1.0 - supp

    def fp_cond(c):
        return c[1] > 0

    def fp_body(c):
        kc, _ = c
        krow = t(kc)
        sup2 = jnp.any(mat & (krow > 0.5), axis=1, keepdims=True)
        new = jnp.where(sup2, 0.0, ext)
        changed = jnp.sum(jnp.abs(new - kc)).astype(jnp.int32)
        return new, changed

    kc, _ = lax.while_loop(fp_cond, fp_body, (ext, jnp.int32(1)))

    out_ref[0] = kc

    n_k = jnp.sum(kc).astype(jnp.int32)
    pos = lax.dot_general(ltri, kc, (((1,), (0,)), ((), ())),
                          preferred_element_type=jnp.float32, precision=lax.Precision.HIGHEST)
    pos_i = pos.astype(jnp.int32) - 1
    cmod = lax.rem(cur, B)
    tgt = lax.rem(pos_i + cmod, B)
    lane = lax.broadcasted_iota(jnp.int32, (B, B), 1)
    perm = ((tgt == lane) & (kc > 0.5)).astype(jnp.float32)

    def compact(row_vals):
        return lax.dot_general(row_vals, perm, (((1,), (0,)), ((), ())),
                               preferred_element_type=jnp.float32, precision=lax.Precision.HIGHEST)

    vst = compact(dst)
    ven = compact(den)
    vpk = compact(dpk)
    vht = compact(dht)
    q = cur // B
    lane_row = lax.broadcasted_iota(jnp.int32, (1, B), 1)
    m_lo = lane_row >= cmod
    m_hi = lane_row < (cmod + n_k - B)

    def rmw(ref, v):
        lo = ref[pl.ds(q, 1), :]
        ref[pl.ds(q, 1), :] = jnp.where(m_lo, v, lo)
        hi = ref[pl.ds(q + 1, 1), :]
        ref[pl.ds(q + 1, 1), :] = jnp.where(m_hi, v, hi)

    rmw(cst_ref, vst)
    rmw(cen_ref, ven)
    rmw(cpk_ref, vpk)
    rmw(cht_ref, vht)
    cur_ref[0] = cur + n_k


def _nms_keep(st, en, pk, ht, interpret=False):
    K, B = st.shape
    stT = st.reshape(K, B, 1)
    enT = en.reshape(K, B, 1)
    pkT = pk.reshape(K, B, 1)
    htT = ht.reshape(K, B, 1)
    full = pl.BlockSpec((K, B), lambda k: (0, 0))
    colb = pl.BlockSpec((1, B, 1), lambda k: (k, 0, 0))
    keep = pl.pallas_call(
        _nms_body,
        grid=(K,),
        in_specs=[full, full, full, full, colb, colb, colb, colb],
        out_specs=colb,
        out_shape=jax.ShapeDtypeStruct((K, B, 1), jnp.float32),
        scratch_shapes=[pltpu.VMEM((K + 2, B), jnp.float32)] * 4
        + [pltpu.SMEM((1,), jnp.int32)],
        interpret=interpret,
    )(st, en, pk, ht, stT, enT, pkT, htT)
    return keep.reshape(K * B)


def _run(output, interpret=False):
    n, c = output.shape
    order = jnp.argsort(-output[:, 0])
    boxes = output[order]
    K = -(-n // _B)
    npad = K * _B
    pad = jnp.zeros((npad - n, c), jnp.float32)
    pad = pad.at[:, 1].set(1e9).at[:, 2].set(2e9).at[:, 3].set(1.5e9)
    pad = pad.at[:, 4].set(1.0)
    bp = jnp.concatenate([boxes, pad], axis=0)
    st = bp[:, 1].reshape(K, _B)
    en = bp[:, 2].reshape(K, _B)
    pk = bp[:, 3].reshape(K, _B)
    ht = bp[:, 4].reshape(K, _B)
    keep = _nms_keep(st, en, pk, ht, interpret=interpret)[:n]
    return boxes[:, 1:] * keep[:, None]


def kernel(output):
    return _run(output)
